# Initial kernel scaffold; baseline (speedup 1.0000x reference)
import jax
import jax.numpy as jnp
from jax.experimental import pallas as pl


def kernel(trip_od, src_embedding, dst_embedding, distm, w, b):
    raise NotImplementedError("write your pallas kernel here")


# speedup vs baseline: 10.8002x; 10.8002x over previous
import functools

import jax
import jax.numpy as jnp
from jax import lax
from jax.experimental import pallas as pl
from jax.experimental.pallas import tpu as pltpu
from jax.experimental.pallas import tpu_sc as plsc

N_NODES = 5000
EMB = 64
N_EDGES = 1_000_000
EPAD = 1_048_576
NODE_PAD = 5120
NC, NS, LANES = 2, 16, 16
NW = NC * NS
PER_TILE = EPAD // NW
CHUNK = 2048
NCHUNKS = PER_TILE // CHUNK
ROWS = CHUNK // 128


def _precompute_body(src_ref, dst_ref, wa_ref, wc_ref, dist_ref,
                     node_ref, maxd_ref):
    g = pl.program_id(0)

    @pl.when(g == 0)
    def _():
        dims = (((1,), (1,)), ((), ()))
        a_row = lax.dot_general(wa_ref[...], src_ref[...], dims,
                                preferred_element_type=jnp.float32)
        c_row = lax.dot_general(wc_ref[...], dst_ref[...], dims,
                                preferred_element_type=jnp.float32)
        rs = jnp.max(src_ref[...], axis=1)[None, :]
        rd = jnp.max(dst_ref[...], axis=1)[None, :]
        node_ref[...] = jnp.concatenate([a_row, c_row, rs, rd], axis=0)
        maxd_ref[...] = jnp.full((1, 1), -jnp.inf, jnp.float32)

    blk_max = jnp.max(dist_ref[...]).reshape(1, 1)
    maxd_ref[...] = jnp.maximum(maxd_ref[...], blk_max)


def _sc_body(i_hbm, j_hbm, distm_hbm, node_hbm, s_hbm, d_hbm, maxes_hbm,
             a_v, c_v, rs_v, rd_v, iv, jv, fv, sv, dv, mv, sem):
    wid = lax.axis_index("s") * NC + lax.axis_index("c")
    base = wid * PER_TILE

    pltpu.sync_copy(node_hbm.at[0], a_v)
    pltpu.sync_copy(node_hbm.at[1], c_v)
    pltpu.sync_copy(node_hbm.at[2], rs_v)
    pltpu.sync_copy(node_hbm.at[3], rd_v)
    mv[...] = jnp.full((LANES,), -jnp.inf, jnp.float32)

    def chunk_step(t, carry):
        off = base + t * CHUNK
        pltpu.sync_copy(i_hbm.at[pl.ds(off, CHUNK)], iv)
        pltpu.sync_copy(j_hbm.at[pl.ds(off, CHUNK)], jv)
        for r in range(ROWS):
            for g in range(128 // LANES):
                o = r * 128 + g * LANES
                ii = iv[pl.ds(o, LANES)]
                jj = jv[pl.ds(o, LANES)]
                sv[pl.ds(o, LANES)] = (plsc.load_gather(a_v, [ii]) +
                                       plsc.load_gather(c_v, [jj]))
                mg = jnp.maximum(plsc.load_gather(rs_v, [ii]),
                                 plsc.load_gather(rd_v, [jj]))
                mv[...] = jnp.maximum(mv[...], mg)
                fv[r, pl.ds(g * LANES, LANES)] = ii * N_NODES + jj
        copies = [pltpu.make_async_copy(distm_hbm.at[fv.at[r]],
                                        dv.at[pl.ds(r * 128, 128)], sem)
                  for r in range(ROWS)]
        for cp in copies:
            cp.start()
        for cp in copies:
            cp.wait()
        pltpu.sync_copy(sv, s_hbm.at[pl.ds(off, CHUNK)])
        pltpu.sync_copy(dv, d_hbm.at[pl.ds(off, CHUNK)])
        return carry

    lax.fori_loop(0, NCHUNKS, chunk_step, 0)
    pltpu.sync_copy(mv, maxes_hbm.at[wid])


def _combine_body(s_ref, d_ref, maxes_ref, maxd_ref, wmid_ref, b_ref,
                  out_ref):
    kscale = jnp.max(maxes_ref[...]) * wmid_ref[0, 0] / maxd_ref[0, 0]
    out_ref[...] = s_ref[...] + kscale * d_ref[...] + b_ref[0, 0]


@jax.jit
def kernel(trip_od, src_embedding, dst_embedding, distm, w, b):
    trip = trip_od.astype(jnp.int32)
    npad = EPAD - N_EDGES
    iv = jnp.concatenate([trip[:, 0], jnp.broadcast_to(trip[0, 0], (npad,))])
    jv = jnp.concatenate([trip[:, 1], jnp.broadcast_to(trip[0, 1], (npad,))])
    src_p = jnp.pad(src_embedding, ((0, NODE_PAD - N_NODES), (0, 0)))
    dst_p = jnp.pad(dst_embedding, ((0, NODE_PAD - N_NODES), (0, 0)))
    wa = w[:EMB].reshape(1, EMB)
    wc = w[EMB + 1:].reshape(1, EMB)
    wmid = w[EMB].reshape(1, 1)
    b2 = b.reshape(1, 1)
    distm_flat = distm.reshape(-1)

    rows_blk = 200
    grid = N_NODES // rows_blk
    node_tab, maxd = pl.pallas_call(
        _precompute_body,
        grid=(grid,),
        in_specs=[
            pl.BlockSpec((NODE_PAD, EMB), lambda g: (0, 0)),
            pl.BlockSpec((NODE_PAD, EMB), lambda g: (0, 0)),
            pl.BlockSpec((1, EMB), lambda g: (0, 0)),
            pl.BlockSpec((1, EMB), lambda g: (0, 0)),
            pl.BlockSpec((rows_blk, N_NODES), lambda g: (g, 0)),
        ],
        out_specs=[
            pl.BlockSpec((4, NODE_PAD), lambda g: (0, 0)),
            pl.BlockSpec((1, 1), lambda g: (0, 0)),
        ],
        out_shape=[
            jax.ShapeDtypeStruct((4, NODE_PAD), jnp.float32),
            jax.ShapeDtypeStruct((1, 1), jnp.float32),
        ],
    )(src_p, dst_p, wa, wc, distm)

    sc_kernel = functools.partial(
        pl.kernel,
        out_type=(
            jax.ShapeDtypeStruct((EPAD,), jnp.float32),
            jax.ShapeDtypeStruct((EPAD,), jnp.float32),
            jax.ShapeDtypeStruct((NW, LANES), jnp.float32),
        ),
        mesh=plsc.VectorSubcoreMesh(core_axis_name="c", subcore_axis_name="s"),
        compiler_params=pltpu.CompilerParams(needs_layout_passes=False),
        scratch_types=[
            pltpu.VMEM((NODE_PAD,), jnp.float32),
            pltpu.VMEM((NODE_PAD,), jnp.float32),
            pltpu.VMEM((NODE_PAD,), jnp.float32),
            pltpu.VMEM((NODE_PAD,), jnp.float32),
            pltpu.VMEM((CHUNK,), jnp.int32),
            pltpu.VMEM((CHUNK,), jnp.int32),
            pltpu.VMEM((ROWS, 128), jnp.int32),
            pltpu.VMEM((CHUNK,), jnp.float32),
            pltpu.VMEM((CHUNK,), jnp.float32),
            pltpu.VMEM((LANES,), jnp.float32),
            pltpu.SemaphoreType.DMA,
        ],
    )(_sc_body)
    s_e, d_e, tile_maxes = sc_kernel(iv, jv, distm_flat, node_tab)

    s2 = s_e.reshape(1024, 1024)
    d2 = d_e.reshape(1024, 1024)
    out2 = pl.pallas_call(
        _combine_body,
        grid=(8,),
        in_specs=[
            pl.BlockSpec((128, 1024), lambda g: (g, 0)),
            pl.BlockSpec((128, 1024), lambda g: (g, 0)),
            pl.BlockSpec((NW, LANES), lambda g: (0, 0)),
            pl.BlockSpec((1, 1), lambda g: (0, 0)),
            pl.BlockSpec((1, 1), lambda g: (0, 0)),
            pl.BlockSpec((1, 1), lambda g: (0, 0)),
        ],
        out_specs=pl.BlockSpec((128, 1024), lambda g: (g, 0)),
        out_shape=jax.ShapeDtypeStruct((1024, 1024), jnp.float32),
    )(s2, d2, tile_maxes, maxd, wmid, b2)

    return out2.reshape(-1)[:N_EDGES]


# speedup vs baseline: 11.9148x; 1.1032x over previous
import functools

import jax
import jax.numpy as jnp
from jax import lax
from jax.experimental import pallas as pl
from jax.experimental.pallas import tpu as pltpu
from jax.experimental.pallas import tpu_sc as plsc

N_NODES = 5000
EMB = 64
N_EDGES = 1_000_000
EPAD = 1_048_576
NC, NS, LANES = 2, 16, 16
NW = NC * NS
PER_TILE = EPAD // NW
CHUNK = 2048
NCHUNKS = PER_TILE // CHUNK


def _node_tab_body(src_ref, dst_ref, wa_ref, wc_ref, node_ref):
    dims = (((1,), (1,)), ((), ()))
    a_row = lax.dot_general(wa_ref[...], src_ref[...], dims,
                            preferred_element_type=jnp.float32)
    c_row = lax.dot_general(wc_ref[...], dst_ref[...], dims,
                            preferred_element_type=jnp.float32)
    rs = jnp.max(src_ref[...], axis=1)[None, :]
    rd = jnp.max(dst_ref[...], axis=1)[None, :]
    node_ref[...] = jnp.concatenate([a_row, c_row, rs, rd], axis=0)


def _maxd_body(dist_ref, maxd_ref):
    g = pl.program_id(0)

    @pl.when(g == 0)
    def _():
        maxd_ref[...] = jnp.full((1, 1), -jnp.inf, jnp.float32)

    blk_max = jnp.max(dist_ref[...]).reshape(1, 1)
    maxd_ref[...] = jnp.maximum(maxd_ref[...], blk_max)


def _sc_body(i_hbm, j_hbm, distm_hbm, node_hbm, s_hbm, d_hbm, maxes_hbm,
             a_v, c_v, rs_v, rd_v, iv0, iv1, jv0, jv1, fv0, fv1,
             sv0, sv1, dv0, dv1, mv, sem_i, sem_j, sem_g, sem_s, sem_d):
    iv, jv, fv = (iv0, iv1), (jv0, jv1), (fv0, fv1)
    sv, dv = (sv0, sv1), (dv0, dv1)
    wid = lax.axis_index("s") * NC + lax.axis_index("c")
    base = wid * PER_TILE

    pltpu.sync_copy(node_hbm.at[0], a_v)
    pltpu.sync_copy(node_hbm.at[1], c_v)
    pltpu.sync_copy(node_hbm.at[2], rs_v)
    pltpu.sync_copy(node_hbm.at[3], rd_v)

    def idx_copies(t, b):
        off = base + t * CHUNK
        return (pltpu.make_async_copy(i_hbm.at[pl.ds(off, CHUNK)], iv[b],
                                      sem_i.at[b]),
                pltpu.make_async_copy(j_hbm.at[pl.ds(off, CHUNK)], jv[b],
                                      sem_j.at[b]))

    def gather_copy(b):
        return pltpu.make_async_copy(distm_hbm.at[fv[b]], dv[b],
                                     sem_g.at[b])

    def store_copies(t, b):
        off = base + t * CHUNK
        return (pltpu.make_async_copy(sv[b], s_hbm.at[pl.ds(off, CHUNK)],
                                      sem_s.at[b]),
                pltpu.make_async_copy(dv[b], d_hbm.at[pl.ds(off, CHUNK)],
                                      sem_d.at[b]))

    for cp in idx_copies(0, 0):
        cp.start()

    def step(t, b, m):
        nt = t + 1

        @pl.when(nt < NCHUNKS)
        def _():
            for cp in idx_copies(nt, 1 - b):
                cp.start()

        for cp in idx_copies(t, b):
            cp.wait()

        @pl.when(t >= 2)
        def _():
            for cp in store_copies(t - 2, b):
                cp.wait()

        for g in range(CHUNK // LANES):
            o = g * LANES
            ii = iv[b][pl.ds(o, LANES)]
            jj = jv[b][pl.ds(o, LANES)]
            sv[b][pl.ds(o, LANES)] = (plsc.load_gather(a_v, [ii]) +
                                      plsc.load_gather(c_v, [jj]))
            m = jnp.maximum(m, plsc.load_gather(rs_v, [ii]))
            m = jnp.maximum(m, plsc.load_gather(rd_v, [jj]))
            fv[b][pl.ds(o, LANES)] = ii * N_NODES + jj

        @pl.when(t >= 1)
        def _():
            gather_copy(1 - b).wait()
            for cp in store_copies(t - 1, 1 - b):
                cp.start()

        gather_copy(b).start()
        return m

    def outer(p, m):
        m = step(2 * p, 0, m)
        m = step(2 * p + 1, 1, m)
        return m

    m = lax.fori_loop(0, NCHUNKS // 2, outer,
                      jnp.full((LANES,), -jnp.inf, jnp.float32))

    last = (NCHUNKS - 1) % 2
    gather_copy(last).wait()
    for cp in store_copies(NCHUNKS - 1, last):
        cp.start()
    for b in (1 - last, last):
        for cp in store_copies(NCHUNKS - 2 + (b == last), b):
            cp.wait()

    mv[...] = m
    pltpu.sync_copy(mv, maxes_hbm.at[wid])


def _combine_body(s_ref, d_ref, maxes_ref, maxd_ref, wmid_ref, b_ref,
                  out_ref):
    kscale = jnp.max(maxes_ref[...]) * wmid_ref[0, 0] / maxd_ref[0, 0]
    out_ref[...] = s_ref[...] + kscale * d_ref[...] + b_ref[0, 0]


@jax.jit
def kernel(trip_od, src_embedding, dst_embedding, distm, w, b):
    trip = trip_od.astype(jnp.int32)
    npad = EPAD - N_EDGES
    iv = jnp.concatenate([trip[:, 0], jnp.broadcast_to(trip[0, 0], (npad,))])
    jv = jnp.concatenate([trip[:, 1], jnp.broadcast_to(trip[0, 1], (npad,))])
    wa = w[:EMB].reshape(1, EMB)
    wc = w[EMB + 1:].reshape(1, EMB)
    wmid = w[EMB].reshape(1, 1)
    b2 = b.reshape(1, 1)
    distm_flat = distm.reshape(-1)

    node_tab = pl.pallas_call(
        _node_tab_body,
        in_specs=[
            pl.BlockSpec((N_NODES, EMB), lambda: (0, 0)),
            pl.BlockSpec((N_NODES, EMB), lambda: (0, 0)),
            pl.BlockSpec((1, EMB), lambda: (0, 0)),
            pl.BlockSpec((1, EMB), lambda: (0, 0)),
        ],
        out_specs=pl.BlockSpec((4, N_NODES), lambda: (0, 0)),
        out_shape=jax.ShapeDtypeStruct((4, N_NODES), jnp.float32),
    )(src_embedding, dst_embedding, wa, wc)

    sc_kernel = functools.partial(
        pl.kernel,
        out_type=(
            jax.ShapeDtypeStruct((EPAD,), jnp.float32),
            jax.ShapeDtypeStruct((EPAD,), jnp.float32),
            jax.ShapeDtypeStruct((NW, LANES), jnp.float32),
        ),
        mesh=plsc.VectorSubcoreMesh(core_axis_name="c", subcore_axis_name="s"),
        compiler_params=pltpu.CompilerParams(needs_layout_passes=False),
        scratch_types=[
            pltpu.VMEM((N_NODES,), jnp.float32),
            pltpu.VMEM((N_NODES,), jnp.float32),
            pltpu.VMEM((N_NODES,), jnp.float32),
            pltpu.VMEM((N_NODES,), jnp.float32),
            pltpu.VMEM((CHUNK,), jnp.int32),
            pltpu.VMEM((CHUNK,), jnp.int32),
            pltpu.VMEM((CHUNK,), jnp.int32),
            pltpu.VMEM((CHUNK,), jnp.int32),
            pltpu.VMEM((CHUNK,), jnp.int32),
            pltpu.VMEM((CHUNK,), jnp.int32),
            pltpu.VMEM((CHUNK,), jnp.float32),
            pltpu.VMEM((CHUNK,), jnp.float32),
            pltpu.VMEM((CHUNK,), jnp.float32),
            pltpu.VMEM((CHUNK,), jnp.float32),
            pltpu.VMEM((LANES,), jnp.float32),
            pltpu.SemaphoreType.DMA((2,)),
            pltpu.SemaphoreType.DMA((2,)),
            pltpu.SemaphoreType.DMA((2,)),
            pltpu.SemaphoreType.DMA((2,)),
            pltpu.SemaphoreType.DMA((2,)),
        ],
    )(_sc_body)
    s_e, d_e, tile_maxes = sc_kernel(iv, jv, distm_flat, node_tab)

    rows_blk = 200
    maxd = pl.pallas_call(
        _maxd_body,
        grid=(N_NODES // rows_blk,),
        in_specs=[pl.BlockSpec((rows_blk, N_NODES), lambda g: (g, 0))],
        out_specs=pl.BlockSpec((1, 1), lambda g: (0, 0)),
        out_shape=jax.ShapeDtypeStruct((1, 1), jnp.float32),
    )(distm)

    blk = EPAD // 8
    out = pl.pallas_call(
        _combine_body,
        grid=(8,),
        in_specs=[
            pl.BlockSpec((blk,), lambda g: (g,)),
            pl.BlockSpec((blk,), lambda g: (g,)),
            pl.BlockSpec((NW, LANES), lambda g: (0, 0)),
            pl.BlockSpec((1, 1), lambda g: (0, 0)),
            pl.BlockSpec((1, 1), lambda g: (0, 0)),
            pl.BlockSpec((1, 1), lambda g: (0, 0)),
        ],
        out_specs=pl.BlockSpec((blk,), lambda g: (g,)),
        out_shape=jax.ShapeDtypeStruct((EPAD,), jnp.float32),
    )(s_e, d_e, tile_maxes, maxd, wmid, b2)

    return out[:N_EDGES]


# speedup vs baseline: 12.0613x; 1.0123x over previous
import functools

import jax
import jax.numpy as jnp
from jax import lax
from jax.experimental import pallas as pl
from jax.experimental.pallas import tpu as pltpu
from jax.experimental.pallas import tpu_sc as plsc

N_NODES = 5000
EMB = 64
N_EDGES = 1_000_000
EPAD = 1_048_576
NC, NS, LANES = 2, 16, 16
NW = NC * NS
PER_TILE = EPAD // NW
CHUNK = 1024
NCHUNKS = PER_TILE // CHUNK
RING = 4


def _node_tab_body(src_ref, dst_ref, wa_ref, wc_ref, node_ref):
    dims = (((1,), (1,)), ((), ()))
    a_row = lax.dot_general(wa_ref[...], src_ref[...], dims,
                            preferred_element_type=jnp.float32)
    c_row = lax.dot_general(wc_ref[...], dst_ref[...], dims,
                            preferred_element_type=jnp.float32)
    rs = jnp.max(src_ref[...], axis=1)[None, :]
    rd = jnp.max(dst_ref[...], axis=1)[None, :]
    node_ref[...] = jnp.concatenate([a_row, c_row, rs, rd], axis=0)


def _maxd_body(dist_ref, maxd_ref):
    g = pl.program_id(0)

    @pl.when(g == 0)
    def _():
        maxd_ref[...] = jnp.full((1, 1), -jnp.inf, jnp.float32)

    blk_max = jnp.max(dist_ref[...]).reshape(1, 1)
    maxd_ref[...] = jnp.maximum(maxd_ref[...], blk_max)


def _sc_s_body(i_hbm, j_hbm, node_hbm, s_hbm, maxes_hbm,
               a_v, c_v, rs_v, rd_v, iv0, iv1, jv0, jv1, sv0, sv1, mv,
               sem_i, sem_j, sem_s):
    iv, jv, sv = (iv0, iv1), (jv0, jv1), (sv0, sv1)
    wid = lax.axis_index("s") * NC + lax.axis_index("c")
    base = wid * PER_TILE

    pltpu.sync_copy(node_hbm.at[0], a_v)
    pltpu.sync_copy(node_hbm.at[1], c_v)
    pltpu.sync_copy(node_hbm.at[2], rs_v)
    pltpu.sync_copy(node_hbm.at[3], rd_v)

    def idx_copies(t, b):
        off = base + t * CHUNK
        return (pltpu.make_async_copy(i_hbm.at[pl.ds(off, CHUNK)], iv[b],
                                      sem_i.at[b]),
                pltpu.make_async_copy(j_hbm.at[pl.ds(off, CHUNK)], jv[b],
                                      sem_j.at[b]))

    def store_copy(t, b):
        off = base + t * CHUNK
        return pltpu.make_async_copy(sv[b], s_hbm.at[pl.ds(off, CHUNK)],
                                     sem_s.at[b])

    for cp in idx_copies(0, 0):
        cp.start()

    def step(t, b, m):
        @pl.when(t + 1 < NCHUNKS)
        def _():
            for cp in idx_copies(t + 1, 1 - b):
                cp.start()

        for cp in idx_copies(t, b):
            cp.wait()

        @pl.when(t >= 2)
        def _():
            store_copy(t - 2, b).wait()

        for g in range(CHUNK // LANES):
            o = g * LANES
            ii = iv[b][pl.ds(o, LANES)]
            jj = jv[b][pl.ds(o, LANES)]
            sv[b][pl.ds(o, LANES)] = (plsc.load_gather(a_v, [ii]) +
                                      plsc.load_gather(c_v, [jj]))
            m = jnp.maximum(m, plsc.load_gather(rs_v, [ii]))
            m = jnp.maximum(m, plsc.load_gather(rd_v, [jj]))

        store_copy(t, b).start()
        return m

    def outer(p, m):
        m = step(2 * p, 0, m)
        m = step(2 * p + 1, 1, m)
        return m

    m = lax.fori_loop(0, NCHUNKS // 2, outer,
                      jnp.full((LANES,), -jnp.inf, jnp.float32))

    for b in (0, 1):
        store_copy(NCHUNKS - 2 + b, b if NCHUNKS % 2 == 0 else 1 - b).wait()

    mv[...] = m
    pltpu.sync_copy(mv, maxes_hbm.at[wid])


def _sc_d_body(i_hbm, j_hbm, distm_hbm, d_hbm,
               iv0, iv1, iv2, iv3, jv0, jv1, jv2, jv3,
               fv0, fv1, fv2, fv3, dv0, dv1, dv2, dv3,
               sem_i, sem_j, sem_g, sem_d):
    iv, jv = (iv0, iv1, iv2, iv3), (jv0, jv1, jv2, jv3)
    fv, dv = (fv0, fv1, fv2, fv3), (dv0, dv1, dv2, dv3)
    wid = lax.axis_index("s") * NC + lax.axis_index("c")
    base = wid * PER_TILE

    def idx_copies(t, b):
        off = base + t * CHUNK
        return (pltpu.make_async_copy(i_hbm.at[pl.ds(off, CHUNK)], iv[b],
                                      sem_i.at[b]),
                pltpu.make_async_copy(j_hbm.at[pl.ds(off, CHUNK)], jv[b],
                                      sem_j.at[b]))

    def gather_copy(b):
        return pltpu.make_async_copy(distm_hbm.at[fv[b]], dv[b],
                                     sem_g.at[b])

    def store_copy(t, b):
        off = base + t * CHUNK
        return pltpu.make_async_copy(dv[b], d_hbm.at[pl.ds(off, CHUNK)],
                                     sem_d.at[b])

    for t0 in (0, 1):
        for cp in idx_copies(t0, t0):
            cp.start()

    def step(t, b):
        @pl.when(t + 2 < NCHUNKS)
        def _():
            for cp in idx_copies(t + 2, (b + 2) % RING):
                cp.start()

        for cp in idx_copies(t, b):
            cp.wait()

        @pl.when(t >= RING)
        def _():
            store_copy(t - RING, b).wait()

        for g in range(CHUNK // LANES):
            o = g * LANES
            ii = iv[b][pl.ds(o, LANES)]
            jj = jv[b][pl.ds(o, LANES)]
            fv[b][pl.ds(o, LANES)] = ii * N_NODES + jj

        gather_copy(b).start()

        @pl.when(t >= 2)
        def _():
            gather_copy((b - 2) % RING).wait()
            store_copy(t - 2, (b - 2) % RING).start()

    def outer(p, _):
        for q in range(RING):
            step(RING * p + q, q)
        return 0

    lax.fori_loop(0, NCHUNKS // RING, outer, 0)

    for t in (NCHUNKS - 1, NCHUNKS):
        b = (t - 1) % RING
        gather_copy(b).wait()
        store_copy(t - 1, b).start()
    for t in range(NCHUNKS - RING, NCHUNKS):
        store_copy(t, t % RING).wait()


def _combine_body(s_ref, d_ref, maxes_ref, maxd_ref, wmid_ref, b_ref,
                  out_ref):
    kscale = jnp.max(maxes_ref[...]) * wmid_ref[0, 0] / maxd_ref[0, 0]
    out_ref[...] = s_ref[...] + kscale * d_ref[...] + b_ref[0, 0]


@jax.jit
def kernel(trip_od, src_embedding, dst_embedding, distm, w, b):
    trip = trip_od.astype(jnp.int32)
    npad = EPAD - N_EDGES
    iv = jnp.concatenate([trip[:, 0], jnp.broadcast_to(trip[0, 0], (npad,))])
    jv = jnp.concatenate([trip[:, 1], jnp.broadcast_to(trip[0, 1], (npad,))])
    wa = w[:EMB].reshape(1, EMB)
    wc = w[EMB + 1:].reshape(1, EMB)
    wmid = w[EMB].reshape(1, 1)
    b2 = b.reshape(1, 1)

    node_tab = pl.pallas_call(
        _node_tab_body,
        in_specs=[
            pl.BlockSpec((N_NODES, EMB), lambda: (0, 0)),
            pl.BlockSpec((N_NODES, EMB), lambda: (0, 0)),
            pl.BlockSpec((1, EMB), lambda: (0, 0)),
            pl.BlockSpec((1, EMB), lambda: (0, 0)),
        ],
        out_specs=pl.BlockSpec((4, N_NODES), lambda: (0, 0)),
        out_shape=jax.ShapeDtypeStruct((4, N_NODES), jnp.float32),
    )(src_embedding, dst_embedding, wa, wc)

    sc_s = functools.partial(
        pl.kernel,
        out_type=(
            jax.ShapeDtypeStruct((EPAD,), jnp.float32),
            jax.ShapeDtypeStruct((NW, LANES), jnp.float32),
        ),
        mesh=plsc.VectorSubcoreMesh(core_axis_name="c", subcore_axis_name="s"),
        compiler_params=pltpu.CompilerParams(needs_layout_passes=False),
        scratch_types=(
            [pltpu.VMEM((N_NODES,), jnp.float32)] * 4 +
            [pltpu.VMEM((CHUNK,), jnp.int32)] * 4 +
            [pltpu.VMEM((CHUNK,), jnp.float32)] * 2 +
            [pltpu.VMEM((LANES,), jnp.float32)] +
            [pltpu.SemaphoreType.DMA((2,))] * 3
        ),
    )(_sc_s_body)
    s_e, tile_maxes = sc_s(iv, jv, node_tab)

    distm_flat = distm.reshape(-1)

    sc_d = functools.partial(
        pl.kernel,
        out_type=jax.ShapeDtypeStruct((EPAD,), jnp.float32),
        mesh=plsc.VectorSubcoreMesh(core_axis_name="c", subcore_axis_name="s"),
        compiler_params=pltpu.CompilerParams(needs_layout_passes=False),
        scratch_types=(
            [pltpu.VMEM((CHUNK,), jnp.int32)] * 12 +
            [pltpu.VMEM((CHUNK,), jnp.float32)] * 4 +
            [pltpu.SemaphoreType.DMA((RING,))] * 4
        ),
    )(_sc_d_body)
    d_e = sc_d(iv, jv, distm_flat)

    rows_blk = 200
    maxd = pl.pallas_call(
        _maxd_body,
        grid=(N_NODES // rows_blk,),
        in_specs=[pl.BlockSpec((rows_blk, N_NODES), lambda g: (g, 0))],
        out_specs=pl.BlockSpec((1, 1), lambda g: (0, 0)),
        out_shape=jax.ShapeDtypeStruct((1, 1), jnp.float32),
    )(distm)

    blk = EPAD // 8
    out = pl.pallas_call(
        _combine_body,
        grid=(8,),
        in_specs=[
            pl.BlockSpec((blk,), lambda g: (g,)),
            pl.BlockSpec((blk,), lambda g: (g,)),
            pl.BlockSpec((NW, LANES), lambda g: (0, 0)),
            pl.BlockSpec((1, 1), lambda g: (0, 0)),
            pl.BlockSpec((1, 1), lambda g: (0, 0)),
            pl.BlockSpec((1, 1), lambda g: (0, 0)),
        ],
        out_specs=pl.BlockSpec((blk,), lambda g: (g,)),
        out_shape=jax.ShapeDtypeStruct((EPAD,), jnp.float32),
    )(s_e, d_e, tile_maxes, maxd, wmid, b2)

    return out[:N_EDGES]
